# Initial kernel scaffold; baseline (speedup 1.0000x reference)
import jax
import jax.numpy as jnp
from jax.experimental import pallas as pl


def kernel(raw, t, tar, n_mask, W_time, b_time, W_q, b_q, W_k, b_k, W_v, b_v, W_o, b_o):
    raise NotImplementedError("write your pallas kernel here")


# speedup vs baseline: 1.1057x; 1.1057x over previous
import numpy as np
import jax
import jax.numpy as jnp
from jax.experimental import pallas as pl


_B_BLK = 400


def _tgat_body(t_ref, raw_ref, tar_ref, mask_ref, w_ref, bt_ref, u1_ref,
               u2_ref, wv1_ref, wv0_ref, bv_ref, wo_ref, sc_ref, out_ref):
    t = t_ref[...]
    raw = raw_ref[...]
    w = w_ref[0]
    bt = bt_ref[0]
    bblk, n = t.shape

    te = jnp.cos(t[:, :, None] * w[None, None, :] + bt[None, None, :])

    s1 = jnp.sum(te * u1_ref[0][None, None, :], axis=-1)
    s2 = jnp.sum(te * u2_ref[0][None, None, :], axis=-1)

    alpha = sc_ref[0, 0]
    beta = sc_ref[0, 1]
    gamma = sc_ref[0, 2]
    delta = sc_ref[0, 3]
    bo = sc_ref[0, 4]

    tar = tar_ref[...]
    inv_sqrt_d = jnp.float32(1.0 / np.sqrt(128.0))
    scores = (tar * (raw * alpha + s1 + beta)
              + (raw * gamma + s2 + delta)) * inv_sqrt_d
    scores = jnp.where(mask_ref[...] > 0, scores, jnp.float32(-1e9))

    m = jnp.max(scores, axis=1, keepdims=True)
    e = jnp.exp(scores - m)
    attn = e / jnp.sum(e, axis=1, keepdims=True)

    v2 = jnp.dot(te.reshape(bblk * n, 128), wv1_ref[...])
    v = (v2.reshape(bblk, n, 128)
         + raw[:, :, None] * wv0_ref[0][None, None, :]
         + bv_ref[0][None, None, :])
    h = jnp.sum(attn[:, :, None] * v, axis=1)
    out_ref[...] = jnp.dot(h, wo_ref[...]) + bo


@jax.jit
def kernel(raw, t, tar, n_mask, W_time, b_time, W_q, b_q, W_k, b_k, W_v, b_v,
           W_o, b_o):
    B, N, _ = raw.shape
    D = W_time.shape[1]

    w = W_time[0]
    cq = jnp.cos(b_time) @ W_q[1:] + b_q
    wq0 = W_q[0]
    u1 = W_k[1:] @ wq0
    u2 = W_k[1:] @ cq
    scalars = jnp.stack([
        wq0 @ W_k[0], wq0 @ b_k, cq @ W_k[0], cq @ b_k,
        b_o[0], jnp.float32(0.0), jnp.float32(0.0), jnp.float32(0.0),
    ])[None, :]

    t2 = t[..., 0]
    raw2 = raw[..., 0]

    grid = (B // _B_BLK,)
    full = lambda shape: pl.BlockSpec(shape, lambda i: (0,) * len(shape))
    row_blk = lambda n: pl.BlockSpec((_B_BLK, n), lambda i: (i, 0))

    return pl.pallas_call(
        _tgat_body,
        grid=grid,
        in_specs=[
            row_blk(N),
            row_blk(N),
            row_blk(1),
            row_blk(N),
            full((1, D)),
            full((1, D)),
            full((1, D)),
            full((1, D)),
            full((D, D)),
            full((1, D)),
            full((1, D)),
            full((D, 1)),
            full((1, 8)),
        ],
        out_specs=row_blk(1),
        out_shape=jax.ShapeDtypeStruct((B, 1), jnp.float32),
    )(t2, raw2, tar, n_mask, w[None, :], b_time[None, :], u1[None, :],
      u2[None, :], W_v[1:], W_v[0][None, :], b_v[None, :], W_o, scalars)


# speedup vs baseline: 2.5753x; 2.3290x over previous
import numpy as np
import jax
import jax.numpy as jnp
from jax.experimental import pallas as pl


_B_BLK = 400

_COS_COEFFS = (
    9.999999997488e-01, -4.999999984277e-01, 4.166666344248e-02,
    -1.388886298960e-03, 2.480055350705e-05, -2.753480531065e-07,
    2.060362334884e-09, -9.722560100837e-12,
)
_INV_2PI = 0.15915494309189535
_TWO_PI_HI = np.float32(2.0 * np.pi)
_TWO_PI_LO = np.float32(2.0 * np.pi - float(np.float32(2.0 * np.pi)))


def _fast_cos(x):
    k = jnp.round(x * jnp.float32(_INV_2PI))
    r = x - k * _TWO_PI_HI - k * _TWO_PI_LO
    r2 = r * r
    p = jnp.float32(_COS_COEFFS[-1])
    for c in _COS_COEFFS[-2::-1]:
        p = p * r2 + jnp.float32(c)
    return p


def _tgat_body(t_ref, raw_ref, tar_ref, mask_ref, w_ref, bt_ref, wc_ref,
               wv0_ref, bv_ref, wo_ref, sc_ref, out_ref):
    t = t_ref[...]
    raw = raw_ref[...]
    w = w_ref[0]
    bt = bt_ref[0]
    bblk, n = t.shape

    te = _fast_cos(t[:, :, None] * w[None, None, :] + bt[None, None, :])

    vs = jnp.dot(te.reshape(bblk * n, 128), wc_ref[...])
    s1 = vs[:, 128:129].reshape(bblk, n)
    s2 = vs[:, 129:130].reshape(bblk, n)
    v = (vs[:, :128].reshape(bblk, n, 128)
         + raw[:, :, None] * wv0_ref[0][None, None, :]
         + bv_ref[0][None, None, :])

    alpha = sc_ref[0, 0]
    beta = sc_ref[0, 1]
    gamma = sc_ref[0, 2]
    delta = sc_ref[0, 3]
    bo = sc_ref[0, 4]

    tar = tar_ref[...]
    inv_sqrt_d = jnp.float32(1.0 / np.sqrt(128.0))
    scores = (tar * (raw * alpha + s1 + beta)
              + (raw * gamma + s2 + delta)) * inv_sqrt_d
    scores = jnp.where(mask_ref[...] > 0, scores, jnp.float32(-1e9))

    m = jnp.max(scores, axis=1, keepdims=True)
    e = jnp.exp(scores - m)
    attn = e / jnp.sum(e, axis=1, keepdims=True)

    h = jnp.sum(attn[:, :, None] * v, axis=1)
    out_ref[...] = jnp.dot(h, wo_ref[...]) + bo


@jax.jit
def kernel(raw, t, tar, n_mask, W_time, b_time, W_q, b_q, W_k, b_k, W_v, b_v,
           W_o, b_o):
    B, N, _ = raw.shape
    D = W_time.shape[1]

    w = W_time[0]
    cq = jnp.cos(b_time) @ W_q[1:] + b_q
    wq0 = W_q[0]
    u1 = W_k[1:] @ wq0
    u2 = W_k[1:] @ cq
    wcomb = jnp.concatenate(
        [W_v[1:], u1[:, None], u2[:, None], jnp.zeros((D, 126), jnp.float32)],
        axis=1)
    scalars = jnp.stack([
        wq0 @ W_k[0], wq0 @ b_k, cq @ W_k[0], cq @ b_k,
        b_o[0], jnp.float32(0.0), jnp.float32(0.0), jnp.float32(0.0),
    ])[None, :]

    t2 = t[..., 0]
    raw2 = raw[..., 0]

    grid = (B // _B_BLK,)
    full = lambda shape: pl.BlockSpec(shape, lambda i: (0,) * len(shape))
    row_blk = lambda n: pl.BlockSpec((_B_BLK, n), lambda i: (i, 0))

    return pl.pallas_call(
        _tgat_body,
        grid=grid,
        in_specs=[
            row_blk(N),
            row_blk(N),
            row_blk(1),
            row_blk(N),
            full((1, D)),
            full((1, D)),
            full((D, 2 * D)),
            full((1, D)),
            full((1, D)),
            full((D, 1)),
            full((1, 8)),
        ],
        out_specs=row_blk(1),
        out_shape=jax.ShapeDtypeStruct((B, 1), jnp.float32),
    )(t2, raw2, tar, n_mask, w[None, :], b_time[None, :], wcomb,
      W_v[0][None, :], b_v[None, :], W_o, scalars)


# speedup vs baseline: 3.3303x; 1.2932x over previous
import numpy as np
import jax
import jax.numpy as jnp
from jax.experimental import pallas as pl


_B_BLK = 400

_COS_COEFFS = (
    9.9999921078e-01, -1.9738980356e+01, 6.4928657421e+01,
    -8.5271621534e+01, 5.8790492120e+01, -2.1071105628e+01,
)


def _fast_cos2pi(y):
    f = y - jnp.round(y)
    u = f * f
    p = jnp.float32(_COS_COEFFS[-1])
    for c in _COS_COEFFS[-2::-1]:
        p = p * u + jnp.float32(c)
    return p


def _tgat_body(t_ref, raw_ref, tar_ref, mask_ref, w_ref, bt_ref, wc_ref,
               wv0_ref, bv_ref, wo_ref, sc_ref, out_ref):
    t = t_ref[...]
    raw = raw_ref[...]
    w = w_ref[0]
    bt = bt_ref[0]
    bblk, n = t.shape

    te = _fast_cos2pi(t[:, :, None] * w[None, None, :] + bt[None, None, :])

    vs = jnp.dot(te.reshape(bblk * n, 128), wc_ref[...])
    s1 = vs[:, 128:129].reshape(bblk, n)
    s2 = vs[:, 129:130].reshape(bblk, n)
    vte = vs[:, :128].reshape(bblk, n, 128)

    alpha = sc_ref[0, 0]
    beta = sc_ref[0, 1]
    gamma = sc_ref[0, 2]
    delta = sc_ref[0, 3]
    bo = sc_ref[0, 4]

    tar = tar_ref[...]
    inv_sqrt_d = jnp.float32(1.0 / np.sqrt(128.0))
    scores = (tar * (raw * alpha + s1 + beta)
              + (raw * gamma + s2 + delta)) * inv_sqrt_d
    scores = jnp.where(mask_ref[...] > 0, scores, jnp.float32(-1e9))

    m = jnp.max(scores, axis=1, keepdims=True)
    e = jnp.exp(scores - m)
    attn = e / jnp.sum(e, axis=1, keepdims=True)

    ar = jnp.sum(attn * raw, axis=1, keepdims=True)
    h = (jnp.sum(attn[:, :, None] * vte, axis=1)
         + ar * wv0_ref[0][None, :] + bv_ref[0][None, :])
    out_ref[...] = jnp.dot(h, wo_ref[...]) + bo


@jax.jit
def kernel(raw, t, tar, n_mask, W_time, b_time, W_q, b_q, W_k, b_k, W_v, b_v,
           W_o, b_o):
    B, N, _ = raw.shape
    D = W_time.shape[1]

    inv_2pi = jnp.float32(1.0 / (2.0 * np.pi))
    w = W_time[0] * inv_2pi
    bt = b_time * inv_2pi
    cq = jnp.cos(b_time) @ W_q[1:] + b_q
    wq0 = W_q[0]
    u1 = W_k[1:] @ wq0
    u2 = W_k[1:] @ cq
    wcomb = jnp.concatenate(
        [W_v[1:], u1[:, None], u2[:, None], jnp.zeros((D, 126), jnp.float32)],
        axis=1)
    scalars = jnp.stack([
        wq0 @ W_k[0], wq0 @ b_k, cq @ W_k[0], cq @ b_k,
        b_o[0], jnp.float32(0.0), jnp.float32(0.0), jnp.float32(0.0),
    ])[None, :]

    t2 = t[..., 0]
    raw2 = raw[..., 0]

    grid = (B // _B_BLK,)
    full = lambda shape: pl.BlockSpec(shape, lambda i: (0,) * len(shape))
    row_blk = lambda n: pl.BlockSpec((_B_BLK, n), lambda i: (i, 0))

    return pl.pallas_call(
        _tgat_body,
        grid=grid,
        in_specs=[
            row_blk(N),
            row_blk(N),
            row_blk(1),
            row_blk(N),
            full((1, D)),
            full((1, D)),
            full((D, 2 * D)),
            full((1, D)),
            full((1, D)),
            full((D, 1)),
            full((1, 8)),
        ],
        out_specs=row_blk(1),
        out_shape=jax.ShapeDtypeStruct((B, 1), jnp.float32),
    )(t2, raw2, tar, n_mask, w[None, :], bt[None, :], wcomb,
      W_v[0][None, :], b_v[None, :], W_o, scalars)


# speedup vs baseline: 3.3685x; 1.0115x over previous
import numpy as np
import jax
import jax.numpy as jnp
from jax.experimental import pallas as pl


_B_BLK = 1000

_COS_COEFFS = (
    9.9999921078e-01, -1.9738980356e+01, 6.4928657421e+01,
    -8.5271621534e+01, 5.8790492120e+01, -2.1071105628e+01,
)


def _fast_cos2pi(y):
    f = y - jnp.round(y)
    u = f * f
    p = jnp.float32(_COS_COEFFS[-1])
    for c in _COS_COEFFS[-2::-1]:
        p = p * u + jnp.float32(c)
    return p


_N_CHUNKS = 1


def _tgat_body(t_ref, raw_ref, tar_ref, mask_ref, w_ref, bt_ref, wc_ref,
               wv0_ref, bv_ref, wo_ref, sc_ref, out_ref):
    w = w_ref[0]
    bt = bt_ref[0]
    bblk, n = t_ref.shape
    rows = bblk // _N_CHUNKS

    alpha = sc_ref[0, 0]
    beta = sc_ref[0, 1]
    gamma = sc_ref[0, 2]
    delta = sc_ref[0, 3]
    bo = sc_ref[0, 4]
    inv_sqrt_d = jnp.float32(1.0 / np.sqrt(128.0))

    for c in range(_N_CHUNKS):
        sl = slice(c * rows, (c + 1) * rows)
        t = t_ref[sl, :]
        raw = raw_ref[sl, :]

        te = _fast_cos2pi(t[:, :, None] * w[None, None, :]
                          + bt[None, None, :])

        vs = jnp.dot(te.reshape(rows * n, 128), wc_ref[...])
        s1 = vs[:, 128:129].reshape(rows, n)
        s2 = vs[:, 129:130].reshape(rows, n)
        vte = vs[:, :128].reshape(rows, n, 128)

        tar = tar_ref[sl, :]
        scores = (tar * (raw * alpha + s1 + beta)
                  + (raw * gamma + s2 + delta)) * inv_sqrt_d
        scores = jnp.where(mask_ref[sl, :] > 0, scores, jnp.float32(-1e9))

        m = jnp.max(scores, axis=1, keepdims=True)
        e = jnp.exp(scores - m)
        attn = e / jnp.sum(e, axis=1, keepdims=True)

        ar = jnp.sum(attn * raw, axis=1, keepdims=True)
        h = (jnp.sum(attn[:, :, None] * vte, axis=1)
             + ar * wv0_ref[0][None, :] + bv_ref[0][None, :])
        out_ref[sl, :] = jnp.dot(h, wo_ref[...]) + bo


@jax.jit
def kernel(raw, t, tar, n_mask, W_time, b_time, W_q, b_q, W_k, b_k, W_v, b_v,
           W_o, b_o):
    B, N, _ = raw.shape
    D = W_time.shape[1]

    inv_2pi = jnp.float32(1.0 / (2.0 * np.pi))
    w = W_time[0] * inv_2pi
    bt = b_time * inv_2pi
    cq = jnp.cos(b_time) @ W_q[1:] + b_q
    wq0 = W_q[0]
    u1 = W_k[1:] @ wq0
    u2 = W_k[1:] @ cq
    wcomb = jnp.concatenate(
        [W_v[1:], u1[:, None], u2[:, None], jnp.zeros((D, 126), jnp.float32)],
        axis=1)
    scalars = jnp.stack([
        wq0 @ W_k[0], wq0 @ b_k, cq @ W_k[0], cq @ b_k,
        b_o[0], jnp.float32(0.0), jnp.float32(0.0), jnp.float32(0.0),
    ])[None, :]

    t2 = t[..., 0]
    raw2 = raw[..., 0]

    grid = (B // _B_BLK,)
    full = lambda shape: pl.BlockSpec(shape, lambda i: (0,) * len(shape))
    row_blk = lambda n: pl.BlockSpec((_B_BLK, n), lambda i: (i, 0))

    return pl.pallas_call(
        _tgat_body,
        grid=grid,
        in_specs=[
            row_blk(N),
            row_blk(N),
            row_blk(1),
            row_blk(N),
            full((1, D)),
            full((1, D)),
            full((D, 2 * D)),
            full((1, D)),
            full((1, D)),
            full((D, 1)),
            full((1, 8)),
        ],
        out_specs=row_blk(1),
        out_shape=jax.ShapeDtypeStruct((B, 1), jnp.float32),
    )(t2, raw2, tar, n_mask, w[None, :], bt[None, :], wcomb,
      W_v[0][None, :], b_v[None, :], W_o, scalars)


# speedup vs baseline: 4.1807x; 1.2411x over previous
import numpy as np
import jax
import jax.numpy as jnp
from jax.experimental import pallas as pl


_B_BLK = 1000

_COS_COEFFS = (
    9.9995902084e-01, -1.9730942367e+01, 6.4671441776e+01,
    -8.2390806312e+01, 4.5621051103e+01,
)


def _fast_cos2pi(y):
    f = y - jnp.round(y)
    u = f * f
    p = jnp.float32(_COS_COEFFS[-1])
    for c in _COS_COEFFS[-2::-1]:
        p = p * u + jnp.float32(c)
    return p


_N_CHUNKS = 1


def _tgat_body(t_ref, raw_ref, tar_ref, mask_ref, w_ref, bt_ref, wc_ref,
               u1_ref, u2_ref, wv0_ref, bv_ref, wo_ref, sc_ref, out_ref):
    w = w_ref[0]
    bt = bt_ref[0]
    bblk, n = t_ref.shape
    rows = bblk // _N_CHUNKS

    alpha = sc_ref[0, 0]
    beta = sc_ref[0, 1]
    gamma = sc_ref[0, 2]
    delta = sc_ref[0, 3]
    bo = sc_ref[0, 4]
    inv_sqrt_d = jnp.float32(1.0 / np.sqrt(128.0))

    for c in range(_N_CHUNKS):
        sl = slice(c * rows, (c + 1) * rows)
        t = t_ref[sl, :]
        raw = raw_ref[sl, :]

        te = _fast_cos2pi(t[:, :, None] * w[None, None, :]
                          + bt[None, None, :])

        vs = jnp.dot(te.reshape(rows * n, 128), wc_ref[...])
        vte = vs[:, :128].reshape(rows, n, 128)
        s12 = jnp.swapaxes(vs[:, 128:130].reshape(rows, n, 2), 1, 2)
        s1 = s12[:, 0, :]
        s2 = s12[:, 1, :]

        tar = tar_ref[sl, :]
        scores = (tar * (raw * alpha + s1 + beta)
                  + (raw * gamma + s2 + delta)) * inv_sqrt_d
        scores = jnp.where(mask_ref[sl, :] > 0, scores, jnp.float32(-1e9))

        m = jnp.max(scores, axis=1, keepdims=True)
        e = jnp.exp(scores - m)
        attn = e / jnp.sum(e, axis=1, keepdims=True)

        ar = jnp.sum(attn * raw, axis=1, keepdims=True)
        h = (jnp.sum(attn[:, :, None] * vte, axis=1)
             + ar * wv0_ref[0][None, :] + bv_ref[0][None, :])
        out_ref[sl, :] = jnp.dot(h, wo_ref[...]) + bo


@jax.jit
def kernel(raw, t, tar, n_mask, W_time, b_time, W_q, b_q, W_k, b_k, W_v, b_v,
           W_o, b_o):
    B, N, _ = raw.shape
    D = W_time.shape[1]

    inv_2pi = jnp.float32(1.0 / (2.0 * np.pi))
    w = W_time[0] * inv_2pi
    bt = b_time * inv_2pi
    cq = jnp.cos(b_time) @ W_q[1:] + b_q
    wq0 = W_q[0]
    u1 = W_k[1:] @ wq0
    u2 = W_k[1:] @ cq
    wcomb = jnp.concatenate(
        [W_v[1:], u1[:, None], u2[:, None], jnp.zeros((D, 126), jnp.float32)],
        axis=1)
    scalars = jnp.stack([
        wq0 @ W_k[0], wq0 @ b_k, cq @ W_k[0], cq @ b_k,
        b_o[0], jnp.float32(0.0), jnp.float32(0.0), jnp.float32(0.0),
    ])[None, :]

    t2 = t[..., 0]
    raw2 = raw[..., 0]

    grid = (B // _B_BLK,)
    full = lambda shape: pl.BlockSpec(shape, lambda i: (0,) * len(shape))
    row_blk = lambda n: pl.BlockSpec((_B_BLK, n), lambda i: (i, 0))

    return pl.pallas_call(
        _tgat_body,
        grid=grid,
        in_specs=[
            row_blk(N),
            row_blk(N),
            row_blk(1),
            row_blk(N),
            full((1, D)),
            full((1, D)),
            full((D, 2 * D)),
            full((1, D)),
            full((1, D)),
            full((1, D)),
            full((1, D)),
            full((D, 1)),
            full((1, 8)),
        ],
        out_specs=row_blk(1),
        out_shape=jax.ShapeDtypeStruct((B, 1), jnp.float32),
    )(t2, raw2, tar, n_mask, w[None, :], bt[None, :], wcomb,
      u1[None, :], u2[None, :], W_v[0][None, :], b_v[None, :], W_o, scalars)
